# Initial kernel scaffold; baseline (speedup 1.0000x reference)
import jax
import jax.numpy as jnp
from jax.experimental import pallas as pl


def kernel(x, edge_index, edge_feats, W11, b11, Wc1, bc1, W12, b12, W13, b13, W21, b21, Wc2, bc2, W22, b22, W23, b23, W2, b2):
    raise NotImplementedError("write your pallas kernel here")


# speedup vs baseline: 6.8726x; 6.8726x over previous
import functools

import jax
import jax.numpy as jnp
from jax import lax
from jax.experimental import pallas as pl
from jax.experimental.pallas import tpu as pltpu
from jax.experimental.pallas import tpu_sc as plsc

N_NODES = 50000
N_EDGES = 800000
N_SUBCORES = 16
CHUNK = 768
SUB = CHUNK // 128
GRPS = CHUNK // 16
EDGES_PER_TILE = ((N_EDGES // N_SUBCORES + CHUNK - 1) // CHUNK) * CHUNK
E_PAD = EDGES_PER_TILE * N_SUBCORES
ROWS_MAIN = 3128
ROWS_LAST = N_NODES - 15 * ROWS_MAIN
BN = 2000
GRID = N_NODES // BN


def _lane_bcast(v, i):
    return lax.gather(
        v, jnp.full((16, 1), i, jnp.int32),
        lax.GatherDimensionNumbers(offset_dims=(), collapsed_slice_dims=(0,),
                                   start_index_map=(0,)),
        (1,), mode=lax.GatherScatterMode.PROMISE_IN_BOUNDS)


def _sc_conv_body(y_hbm, src_hbm, dst_hbm, e_hbm, zero_hbm, out_hbm,
                  srcw, dstw, ew, rows, acc, gsem, ssem):
    cid = lax.axis_index("c")
    sid = lax.axis_index("s")
    ytab = y_hbm.at[cid]

    @pl.when(sid < 15)
    def _():
        pltpu.sync_copy(zero_hbm.at[pl.ds(sid * ROWS_MAIN, ROWS_MAIN)],
                        acc.at[pl.ds(sid * ROWS_MAIN, ROWS_MAIN)])

    @pl.when(sid == 15)
    def _():
        pltpu.sync_copy(zero_hbm.at[pl.ds(15 * ROWS_MAIN, ROWS_LAST)],
                        acc.at[pl.ds(15 * ROWS_MAIN, ROWS_LAST)])

    plsc.subcore_barrier()

    n_chunks = EDGES_PER_TILE // CHUNK
    iota16 = lax.iota(jnp.int32, 16)

    def chunk_body(ci, carry):
        row0 = sid * (EDGES_PER_TILE // 128) + ci * SUB
        base = sid * EDGES_PER_TILE + ci * CHUNK
        pltpu.sync_copy(src_hbm.at[pl.ds(row0, SUB)], srcw)
        pltpu.sync_copy(dst_hbm.at[pl.ds(row0, SUB)], dstw)
        pltpu.sync_copy(e_hbm.at[pl.ds(base, CHUNK)], ew)

        cps = [pltpu.async_copy(ytab.at[srcw.at[j]],
                                rows.at[pl.ds(j * 128, 128)], gsem)
               for j in range(SUB)]
        for c in cps:
            c.wait()

        def mul_body(g, _):
            ev = ew[pl.ds(g * 16, 16)]
            for i in range(16):
                r = g * 16 + i
                eb = _lane_bcast(ev, i)
                rows[r, pl.ds(0, 16)] = rows[r, pl.ds(0, 16)] * eb
                rows[r, pl.ds(16, 16)] = rows[r, pl.ds(16, 16)] * eb
            return 0

        lax.fori_loop(0, GRPS, mul_body, 0)

        scs = [pltpu.async_copy(rows.at[pl.ds(j * 128, 128)],
                                acc.at[dstw.at[j]], ssem, add=True)
               for j in range(SUB)]
        for c in scs:
            c.wait()
        return carry

    lax.fori_loop(0, n_chunks, chunk_body, 0)

    plsc.subcore_barrier()

    @pl.when(sid < 15)
    def _():
        pltpu.sync_copy(acc.at[pl.ds(sid * ROWS_MAIN, ROWS_MAIN)],
                        out_hbm.at[cid].at[pl.ds(sid * ROWS_MAIN, ROWS_MAIN)])

    @pl.when(sid == 15)
    def _():
        pltpu.sync_copy(acc.at[pl.ds(15 * ROWS_MAIN, ROWS_LAST)],
                        out_hbm.at[cid].at[pl.ds(15 * ROWS_MAIN, ROWS_LAST)])


def _sc_conv(y2, src2d, dst2d, e_flat, zeros):
    mesh = plsc.VectorSubcoreMesh(core_axis_name="c", subcore_axis_name="s")
    f = pl.kernel(
        _sc_conv_body,
        out_type=jax.ShapeDtypeStruct((2, N_NODES, 32), jnp.float32),
        mesh=mesh,
        scratch_types=[
            pltpu.VMEM((SUB, 128), jnp.int32),
            pltpu.VMEM((SUB, 128), jnp.int32),
            pltpu.VMEM((CHUNK,), jnp.float32),
            pltpu.VMEM((CHUNK, 32), jnp.float32),
            pltpu.VMEM_SHARED((N_NODES, 32), jnp.float32),
            pltpu.SemaphoreType.DMA,
            pltpu.SemaphoreType.DMA,
        ],
        compiler_params=pltpu.CompilerParams(use_tc_tiling_on_sc=False),
    )
    return f(y2, src2d, dst2d, e_flat, zeros)


def _mm(a, w):
    return jax.lax.dot_general(a, w, (((1,), (0,)), ((), ())),
                               preferred_element_type=jnp.float32)


def _dense1_body(x_ref, W11_ref, b11_ref, W12_ref, b12_ref, W13_ref, b13_ref,
                 Wc1_ref, a_ref, c_ref, y_ref):
    xb = x_ref[...]
    a_ref[...] = jnp.maximum(_mm(xb, W11_ref[...]) + b11_ref[...], 0.0)
    c_ref[...] = (jnp.maximum(_mm(xb, W12_ref[...]) + b12_ref[...], 0.0)
                  * jnp.maximum(_mm(xb, W13_ref[...]) + b13_ref[...], 0.0))
    y = _mm(xb, Wc1_ref[...])
    y_ref[0] = y[:, :32]
    y_ref[1] = y[:, 32:]


def _dense2_body(a_ref, c_ref, h_ref, bc1_ref,
                 W21a_ref, W21b_ref, W21c_ref, b21_ref,
                 W22a_ref, W22b_ref, W22c_ref, b22_ref,
                 W23a_ref, W23b_ref, W23c_ref, b23_ref,
                 Wc2a_ref, Wc2b_ref, Wc2c_ref,
                 p_ref, q_ref, y_ref):
    ab = a_ref[...]
    cb = c_ref[...]
    conv = jnp.maximum(
        jnp.concatenate([h_ref[0], h_ref[1]], axis=1) + bc1_ref[...], 0.0)
    p_ref[...] = jnp.maximum(
        _mm(ab, W21a_ref[...]) + _mm(conv, W21b_ref[...])
        + _mm(cb, W21c_ref[...]) + b21_ref[...], 0.0)
    q_ref[...] = (
        jnp.maximum(_mm(ab, W22a_ref[...]) + _mm(conv, W22b_ref[...])
                    + _mm(cb, W22c_ref[...]) + b22_ref[...], 0.0)
        * jnp.maximum(_mm(ab, W23a_ref[...]) + _mm(conv, W23b_ref[...])
                      + _mm(cb, W23c_ref[...]) + b23_ref[...], 0.0))
    y = (_mm(ab, Wc2a_ref[...]) + _mm(conv, Wc2b_ref[...])
         + _mm(cb, Wc2c_ref[...]))
    y_ref[0] = y[:, :32]
    y_ref[1] = y[:, 32:]


def _dense3_body(p_ref, q_ref, h_ref, bc2_ref,
                 W2a_ref, W2b_ref, W2c_ref, b2_ref, out_ref):
    conv = jnp.maximum(
        jnp.concatenate([h_ref[0], h_ref[1]], axis=1) + bc2_ref[...], 0.0)
    z = (_mm(p_ref[...], W2a_ref[...]) + _mm(conv, W2b_ref[...])
         + _mm(q_ref[...], W2c_ref[...]) + b2_ref[...])
    m = jnp.max(z, axis=1, keepdims=True)
    zs = z - m
    out_ref[...] = zs - jnp.log(jnp.sum(jnp.exp(zs), axis=1, keepdims=True))


def _row_spec(w):
    return pl.BlockSpec((BN, w), lambda i: (i, 0))


def _half_spec():
    return pl.BlockSpec((2, BN, 32), lambda i: (0, i, 0))


def _w_spec(shape):
    return pl.BlockSpec(shape, lambda i: tuple(0 for _ in shape))


def kernel(x, edge_index, edge_feats, W11, b11, Wc1, bc1, W12, b12, W13, b13,
           W21, b21, Wc2, bc2, W22, b22, W23, b23, W2, b2):
    n = x.shape[0]
    e_cnt = edge_index.shape[1]
    assert n == N_NODES and e_cnt == N_EDGES

    pad = E_PAD - e_cnt
    src = jnp.concatenate([edge_index[0], jnp.zeros((pad,), jnp.int32)])
    dst = jnp.concatenate(
        [edge_index[1], jnp.arange(pad, dtype=jnp.int32) % n])
    ew = jnp.concatenate([edge_feats[:, 0], jnp.zeros((pad,), jnp.float32)])
    src2d = src.reshape(-1, 128)
    dst2d = dst.reshape(-1, 128)
    zeros = jnp.zeros((n, 32), jnp.float32)

    W21a, W21b, W21c = W21[:64], W21[64:128], W21[128:]
    W22a, W22b, W22c = W22[:64], W22[64:128], W22[128:]
    W23a, W23b, W23c = W23[:64], W23[64:128], W23[128:]
    Wc2a, Wc2b, Wc2c = Wc2[0][:64], Wc2[0][64:128], Wc2[0][128:]
    W2a, W2b, W2c = W2[:64], W2[64:128], W2[128:]
    b11r, b12r, b13r = b11[None], b12[None], b13[None]
    b21r, b22r, b23r = b21[None], b22[None], b23[None]
    bc1r, bc2r, b2r = bc1[None], bc2[None], b2[None]

    a, c, y0 = pl.pallas_call(
        _dense1_body,
        grid=(GRID,),
        in_specs=[_row_spec(64), _w_spec((64, 64)), _w_spec((1, 64)),
                  _w_spec((64, 16)), _w_spec((1, 16)),
                  _w_spec((64, 16)), _w_spec((1, 16)),
                  _w_spec((64, 64))],
        out_specs=[_row_spec(64), _row_spec(16), _half_spec()],
        out_shape=[jax.ShapeDtypeStruct((n, 64), jnp.float32),
                   jax.ShapeDtypeStruct((n, 16), jnp.float32),
                   jax.ShapeDtypeStruct((2, n, 32), jnp.float32)],
    )(x, W11, b11r, W12, b12r, W13, b13r, Wc1[0])

    h0 = _sc_conv(y0, src2d, dst2d, ew, zeros)

    p, q, y1 = pl.pallas_call(
        _dense2_body,
        grid=(GRID,),
        in_specs=[_row_spec(64), _row_spec(16), _half_spec(), _w_spec((1, 64)),
                  _w_spec((64, 64)), _w_spec((64, 64)), _w_spec((16, 64)),
                  _w_spec((1, 64)),
                  _w_spec((64, 16)), _w_spec((64, 16)), _w_spec((16, 16)),
                  _w_spec((1, 16)),
                  _w_spec((64, 16)), _w_spec((64, 16)), _w_spec((16, 16)),
                  _w_spec((1, 16)),
                  _w_spec((64, 64)), _w_spec((64, 64)), _w_spec((16, 64))],
        out_specs=[_row_spec(64), _row_spec(16), _half_spec()],
        out_shape=[jax.ShapeDtypeStruct((n, 64), jnp.float32),
                   jax.ShapeDtypeStruct((n, 16), jnp.float32),
                   jax.ShapeDtypeStruct((2, n, 32), jnp.float32)],
    )(a, c, h0, bc1r,
      W21a, W21b, W21c, b21r,
      W22a, W22b, W22c, b22r,
      W23a, W23b, W23c, b23r,
      Wc2a, Wc2b, Wc2c)

    h1 = _sc_conv(y1, src2d, dst2d, ew, zeros)

    out = pl.pallas_call(
        _dense3_body,
        grid=(GRID,),
        in_specs=[_row_spec(64), _row_spec(16), _half_spec(), _w_spec((1, 64)),
                  _w_spec((64, 128)), _w_spec((64, 128)), _w_spec((16, 128)),
                  _w_spec((1, 128))],
        out_specs=_row_spec(128),
        out_shape=jax.ShapeDtypeStruct((n, 128), jnp.float32),
    )(p, q, h1, bc2r, W2a, W2b, W2c, b2r)

    return out


# speedup vs baseline: 8.9972x; 1.3091x over previous
import functools

import jax
import jax.numpy as jnp
from jax import lax
from jax.experimental import pallas as pl
from jax.experimental.pallas import tpu as pltpu
from jax.experimental.pallas import tpu_sc as plsc

N_NODES = 50000
N_EDGES = 800000
N_SUBCORES = 16
CHUNK = 384
SUB = CHUNK // 128
N_CHUNKS = 132
EDGES_PER_TILE = N_CHUNKS * CHUNK
E_PAD = EDGES_PER_TILE * N_SUBCORES
E_ALLOC = E_PAD + 2 * CHUNK
ROWS_MAIN = 3128
ROWS_LAST = N_NODES - 15 * ROWS_MAIN
BN = 2000
GRID = N_NODES // BN


def _lane_bcast(v, i):
    return lax.gather(
        v, jnp.full((16, 1), i, jnp.int32),
        lax.GatherDimensionNumbers(offset_dims=(), collapsed_slice_dims=(0,),
                                   start_index_map=(0,)),
        (1,), mode=lax.GatherScatterMode.PROMISE_IN_BOUNDS)


def _sc_conv_body(y_hbm, src_hbm, dst_hbm, e_hbm, zero_hbm, out_hbm,
                  srcw0, srcw1, dstw0, dstw1, sdst0, sdst1, ew0, ew1,
                  rows0, rows1, acc, gsem, ssem, lsem):
    cid = lax.axis_index("c")
    sid = lax.axis_index("s")
    ytab = y_hbm.at[cid]
    srcw = (srcw0, srcw1)
    dstw = (dstw0, dstw1)
    sdst = (sdst0, sdst1)
    ew = (ew0, ew1)
    rows = (rows0, rows1)

    @pl.when(sid < 15)
    def _():
        pltpu.sync_copy(zero_hbm.at[pl.ds(sid * ROWS_MAIN, ROWS_MAIN)],
                        acc.at[pl.ds(sid * ROWS_MAIN, ROWS_MAIN)])

    @pl.when(sid == 15)
    def _():
        pltpu.sync_copy(zero_hbm.at[pl.ds(15 * ROWS_MAIN, ROWS_LAST)],
                        acc.at[pl.ds(15 * ROWS_MAIN, ROWS_LAST)])

    plsc.subcore_barrier()


    def _lin(i, b):
        r0 = sid * (EDGES_PER_TILE // 128) + i * SUB
        base = sid * EDGES_PER_TILE + i * CHUNK
        return ((src_hbm.at[pl.ds(r0, SUB)], srcw[b]),
                (dst_hbm.at[pl.ds(r0, SUB)], dstw[b]),
                (e_hbm.at[pl.ds(base, CHUNK)], ew[b]))

    def issue_L(i, b):
        for s, d in _lin(i, b):
            pltpu.async_copy(s, d, lsem)

    def wait_L(i, b):
        for s, d in _lin(i, b):
            pltpu.make_async_copy(s, d, lsem).wait()

    def issue_G(b):
        for j in range(SUB):
            pltpu.async_copy(ytab.at[srcw[b].at[j]], rows[b].at[j], gsem)

    def wait_G(b):
        for j in range(SUB):
            pltpu.make_async_copy(ytab.at[srcw[b].at[j]], rows[b].at[j],
                                  gsem).wait()

    def issue_S(b):
        for j in range(SUB):
            pltpu.async_copy(rows[b].at[j], acc.at[sdst[b].at[j]], ssem,
                             add=True)

    def wait_S(b):
        for j in range(SUB):
            pltpu.make_async_copy(rows[b].at[j], acc.at[sdst[b].at[j]],
                                  ssem).wait()

    def copy_dst(b):
        for j in range(SUB):
            for t in range(8):
                sdst[b][j, pl.ds(t * 16, 16)] = dstw[b][j, pl.ds(t * 16, 16)]

    def scale(b):
        ewb = ew[b]
        rb = rows[b]
        for j in range(SUB):
            view = rb.at[j]

            def g_body(g, _, j=j, view=view):
                ev = ewb[pl.ds(j * 128 + g * 16, 16)]
                for t in range(16):
                    eb = _lane_bcast(ev, t)
                    r = g * 16 + t
                    view[r, pl.ds(0, 16)] = view[r, pl.ds(0, 16)] * eb
                    view[r, pl.ds(16, 16)] = view[r, pl.ds(16, 16)] * eb
                return 0

            lax.fori_loop(0, 8, g_body, 0)

    def chunk_step(i, b, first=False, do_g=True, do_l=True):
        wait_G(b)
        if not first:
            wait_S(1 - b)
        if do_g:
            wait_L(i + 1, 1 - b)
            issue_G(1 - b)
        scale(b)
        copy_dst(b)
        issue_S(b)
        if do_l:
            issue_L(i + 2, b)

    issue_L(0, 0)
    issue_L(1, 1)
    wait_L(0, 0)
    issue_G(0)
    chunk_step(0, 0, first=True)

    def k_body(k, _):
        chunk_step(2 * k + 1, 1)
        chunk_step(2 * k + 2, 0)
        return 0

    lax.fori_loop(0, (N_CHUNKS - 4) // 2, k_body, 0)
    chunk_step(N_CHUNKS - 3, 1)
    chunk_step(N_CHUNKS - 2, 0, do_l=False)
    chunk_step(N_CHUNKS - 1, 1, do_g=False, do_l=False)
    wait_S(1)

    plsc.subcore_barrier()

    @pl.when(sid < 15)
    def _():
        pltpu.sync_copy(acc.at[pl.ds(sid * ROWS_MAIN, ROWS_MAIN)],
                        out_hbm.at[cid].at[pl.ds(sid * ROWS_MAIN, ROWS_MAIN)])

    @pl.when(sid == 15)
    def _():
        pltpu.sync_copy(acc.at[pl.ds(15 * ROWS_MAIN, ROWS_LAST)],
                        out_hbm.at[cid].at[pl.ds(15 * ROWS_MAIN, ROWS_LAST)])


def _sc_conv(y2, src2d, dst2d, e_flat, zeros):
    mesh = plsc.VectorSubcoreMesh(core_axis_name="c", subcore_axis_name="s")
    f = pl.kernel(
        _sc_conv_body,
        out_type=jax.ShapeDtypeStruct((2, N_NODES, 32), jnp.float32),
        mesh=mesh,
        scratch_types=[
            pltpu.VMEM((SUB, 128), jnp.int32),
            pltpu.VMEM((SUB, 128), jnp.int32),
            pltpu.VMEM((SUB, 128), jnp.int32),
            pltpu.VMEM((SUB, 128), jnp.int32),
            pltpu.VMEM((SUB, 128), jnp.int32),
            pltpu.VMEM((SUB, 128), jnp.int32),
            pltpu.VMEM((CHUNK,), jnp.float32),
            pltpu.VMEM((CHUNK,), jnp.float32),
            pltpu.VMEM((SUB, 128, 32), jnp.float32),
            pltpu.VMEM((SUB, 128, 32), jnp.float32),
            pltpu.VMEM_SHARED((N_NODES, 32), jnp.float32),
            pltpu.SemaphoreType.DMA,
            pltpu.SemaphoreType.DMA,
            pltpu.SemaphoreType.DMA,
        ],
        compiler_params=pltpu.CompilerParams(use_tc_tiling_on_sc=False),
    )
    return f(y2, src2d, dst2d, e_flat, zeros)


def _mm(a, w):
    return jax.lax.dot_general(a, w, (((1,), (0,)), ((), ())),
                               preferred_element_type=jnp.float32)


def _dense1_body(x_ref, W11_ref, b11_ref, W12_ref, b12_ref, W13_ref, b13_ref,
                 Wc1_ref, a_ref, c_ref, y_ref):
    xb = x_ref[...]
    a_ref[...] = jnp.maximum(_mm(xb, W11_ref[...]) + b11_ref[...], 0.0)
    c_ref[...] = (jnp.maximum(_mm(xb, W12_ref[...]) + b12_ref[...], 0.0)
                  * jnp.maximum(_mm(xb, W13_ref[...]) + b13_ref[...], 0.0))
    y = _mm(xb, Wc1_ref[...])
    y_ref[0] = y[:, :32]
    y_ref[1] = y[:, 32:]


def _dense2_body(a_ref, c_ref, h_ref, bc1_ref,
                 W21a_ref, W21b_ref, W21c_ref, b21_ref,
                 W22a_ref, W22b_ref, W22c_ref, b22_ref,
                 W23a_ref, W23b_ref, W23c_ref, b23_ref,
                 Wc2a_ref, Wc2b_ref, Wc2c_ref,
                 p_ref, q_ref, y_ref):
    ab = a_ref[...]
    cb = c_ref[...]
    conv = jnp.maximum(
        jnp.concatenate([h_ref[0], h_ref[1]], axis=1) + bc1_ref[...], 0.0)
    p_ref[...] = jnp.maximum(
        _mm(ab, W21a_ref[...]) + _mm(conv, W21b_ref[...])
        + _mm(cb, W21c_ref[...]) + b21_ref[...], 0.0)
    q_ref[...] = (
        jnp.maximum(_mm(ab, W22a_ref[...]) + _mm(conv, W22b_ref[...])
                    + _mm(cb, W22c_ref[...]) + b22_ref[...], 0.0)
        * jnp.maximum(_mm(ab, W23a_ref[...]) + _mm(conv, W23b_ref[...])
                      + _mm(cb, W23c_ref[...]) + b23_ref[...], 0.0))
    y = (_mm(ab, Wc2a_ref[...]) + _mm(conv, Wc2b_ref[...])
         + _mm(cb, Wc2c_ref[...]))
    y_ref[0] = y[:, :32]
    y_ref[1] = y[:, 32:]


def _dense3_body(p_ref, q_ref, h_ref, bc2_ref,
                 W2a_ref, W2b_ref, W2c_ref, b2_ref, out_ref):
    conv = jnp.maximum(
        jnp.concatenate([h_ref[0], h_ref[1]], axis=1) + bc2_ref[...], 0.0)
    z = (_mm(p_ref[...], W2a_ref[...]) + _mm(conv, W2b_ref[...])
         + _mm(q_ref[...], W2c_ref[...]) + b2_ref[...])
    m = jnp.max(z, axis=1, keepdims=True)
    zs = z - m
    out_ref[...] = zs - jnp.log(jnp.sum(jnp.exp(zs), axis=1, keepdims=True))


def _row_spec(w):
    return pl.BlockSpec((BN, w), lambda i: (i, 0))


def _half_spec():
    return pl.BlockSpec((2, BN, 32), lambda i: (0, i, 0))


def _w_spec(shape):
    return pl.BlockSpec(shape, lambda i: tuple(0 for _ in shape))


def kernel(x, edge_index, edge_feats, W11, b11, Wc1, bc1, W12, b12, W13, b13,
           W21, b21, Wc2, bc2, W22, b22, W23, b23, W2, b2):
    n = x.shape[0]
    e_cnt = edge_index.shape[1]
    assert n == N_NODES and e_cnt == N_EDGES

    pad = E_ALLOC - e_cnt
    src = jnp.concatenate([edge_index[0], jnp.zeros((pad,), jnp.int32)])
    dst = jnp.concatenate(
        [edge_index[1], jnp.arange(pad, dtype=jnp.int32) % n])
    ew = jnp.concatenate([edge_feats[:, 0], jnp.zeros((pad,), jnp.float32)])
    src2d = src.reshape(-1, 128)
    dst2d = dst.reshape(-1, 128)
    zeros = jnp.zeros((n, 32), jnp.float32)

    W21a, W21b, W21c = W21[:64], W21[64:128], W21[128:]
    W22a, W22b, W22c = W22[:64], W22[64:128], W22[128:]
    W23a, W23b, W23c = W23[:64], W23[64:128], W23[128:]
    Wc2a, Wc2b, Wc2c = Wc2[0][:64], Wc2[0][64:128], Wc2[0][128:]
    W2a, W2b, W2c = W2[:64], W2[64:128], W2[128:]
    b11r, b12r, b13r = b11[None], b12[None], b13[None]
    b21r, b22r, b23r = b21[None], b22[None], b23[None]
    bc1r, bc2r, b2r = bc1[None], bc2[None], b2[None]

    a, c, y0 = pl.pallas_call(
        _dense1_body,
        grid=(GRID,),
        in_specs=[_row_spec(64), _w_spec((64, 64)), _w_spec((1, 64)),
                  _w_spec((64, 16)), _w_spec((1, 16)),
                  _w_spec((64, 16)), _w_spec((1, 16)),
                  _w_spec((64, 64))],
        out_specs=[_row_spec(64), _row_spec(16), _half_spec()],
        out_shape=[jax.ShapeDtypeStruct((n, 64), jnp.float32),
                   jax.ShapeDtypeStruct((n, 16), jnp.float32),
                   jax.ShapeDtypeStruct((2, n, 32), jnp.float32)],
    )(x, W11, b11r, W12, b12r, W13, b13r, Wc1[0])

    h0 = _sc_conv(y0, src2d, dst2d, ew, zeros)

    p, q, y1 = pl.pallas_call(
        _dense2_body,
        grid=(GRID,),
        in_specs=[_row_spec(64), _row_spec(16), _half_spec(), _w_spec((1, 64)),
                  _w_spec((64, 64)), _w_spec((64, 64)), _w_spec((16, 64)),
                  _w_spec((1, 64)),
                  _w_spec((64, 16)), _w_spec((64, 16)), _w_spec((16, 16)),
                  _w_spec((1, 16)),
                  _w_spec((64, 16)), _w_spec((64, 16)), _w_spec((16, 16)),
                  _w_spec((1, 16)),
                  _w_spec((64, 64)), _w_spec((64, 64)), _w_spec((16, 64))],
        out_specs=[_row_spec(64), _row_spec(16), _half_spec()],
        out_shape=[jax.ShapeDtypeStruct((n, 64), jnp.float32),
                   jax.ShapeDtypeStruct((n, 16), jnp.float32),
                   jax.ShapeDtypeStruct((2, n, 32), jnp.float32)],
    )(a, c, h0, bc1r,
      W21a, W21b, W21c, b21r,
      W22a, W22b, W22c, b22r,
      W23a, W23b, W23c, b23r,
      Wc2a, Wc2b, Wc2c)

    h1 = _sc_conv(y1, src2d, dst2d, ew, zeros)

    out = pl.pallas_call(
        _dense3_body,
        grid=(GRID,),
        in_specs=[_row_spec(64), _row_spec(16), _half_spec(), _w_spec((1, 64)),
                  _w_spec((64, 128)), _w_spec((64, 128)), _w_spec((16, 128)),
                  _w_spec((1, 128))],
        out_specs=_row_spec(128),
        out_shape=jax.ShapeDtypeStruct((n, 128), jnp.float32),
    )(p, q, h1, bc2r, W2a, W2b, W2c, b2r)

    return out


# speedup vs baseline: 10.6039x; 1.1786x over previous
import functools

import jax
import jax.numpy as jnp
from jax import lax
from jax.experimental import pallas as pl
from jax.experimental.pallas import tpu as pltpu
from jax.experimental.pallas import tpu_sc as plsc

N_NODES = 50000
N_EDGES = 800000
N_SUBCORES = 16
CHUNK = 256
SUB = CHUNK // 128
N_CHUNKS = 196
NBUF = 3
EDGES_PER_TILE = N_CHUNKS * CHUNK
E_PAD = EDGES_PER_TILE * N_SUBCORES
E_ALLOC = E_PAD
ROWS_MAIN = 3128
ROWS_LAST = N_NODES - 15 * ROWS_MAIN
BN = 2000
GRID = N_NODES // BN


def _lane_bcast(v, i):
    return lax.gather(
        v, jnp.full((16, 1), i, jnp.int32),
        lax.GatherDimensionNumbers(offset_dims=(), collapsed_slice_dims=(0,),
                                   start_index_map=(0,)),
        (1,), mode=lax.GatherScatterMode.PROMISE_IN_BOUNDS)


def _sc_conv_body(y_hbm, edata_hbm, e_hbm, zero_hbm, out_hbm,
                  ed0, ed1, ed2, sdst0, sdst1, sdst2, ew0, ew1, ew2,
                  rows0, rows1, rows2, acc, gsem, ssem, lsem):
    cid = lax.axis_index("c")
    sid = lax.axis_index("s")
    ytab = y_hbm.at[cid]
    ed = (ed0, ed1, ed2)
    ew = (ew0, ew1, ew2)
    sdst = (sdst0, sdst1, sdst2)
    rows = (rows0, rows1, rows2)

    @pl.when(sid < 15)
    def _():
        pltpu.sync_copy(zero_hbm.at[pl.ds(sid * ROWS_MAIN, ROWS_MAIN)],
                        acc.at[pl.ds(sid * ROWS_MAIN, ROWS_MAIN)])

    @pl.when(sid == 15)
    def _():
        pltpu.sync_copy(zero_hbm.at[pl.ds(15 * ROWS_MAIN, ROWS_LAST)],
                        acc.at[pl.ds(15 * ROWS_MAIN, ROWS_LAST)])

    plsc.subcore_barrier()


    def _lin(i, b):
        r0 = sid * (EDGES_PER_TILE // 128) + i * SUB
        base = sid * EDGES_PER_TILE + i * CHUNK
        return ((edata_hbm.at[pl.ds(r0, SUB)], ed[b]),
                (e_hbm.at[pl.ds(base, CHUNK)], ew[b]))

    def issue_L(i, b):
        for s, d in _lin(i, b):
            pltpu.async_copy(s, d, lsem)

    def wait_L(i, b):
        for s, d in _lin(i, b):
            pltpu.make_async_copy(s, d, lsem).wait()

    def issue_G(b):
        for j in range(SUB):
            pltpu.async_copy(ytab.at[ed[b].at[j, 0]], rows[b].at[j], gsem)

    def wait_G(b):
        for j in range(SUB):
            pltpu.make_async_copy(ytab.at[ed[b].at[j, 0]], rows[b].at[j],
                                  gsem).wait()

    def issue_S(b):
        for j in range(SUB):
            pltpu.async_copy(rows[b].at[j], acc.at[sdst[b].at[j]], ssem,
                             add=True)

    def wait_S(b):
        for j in range(SUB):
            pltpu.make_async_copy(rows[b].at[j], acc.at[sdst[b].at[j]],
                                  ssem).wait()

    def copy_dst(b):
        for j in range(SUB):
            for t in range(8):
                sdst[b][j, pl.ds(t * 16, 16)] = ed[b][j, 1, pl.ds(t * 16, 16)]

    def scale(b):
        rb = rows[b]
        for j in range(SUB):
            view = rb.at[j]

            def g_body(g, _, j=j, view=view):
                ev = ew[b][pl.ds(j * 128 + g * 16, 16)]
                for t in range(16):
                    eb = _lane_bcast(ev, t)
                    r = g * 16 + t
                    view[r, pl.ds(0, 16)] = view[r, pl.ds(0, 16)] * eb
                    view[r, pl.ds(16, 16)] = view[r, pl.ds(16, 16)] * eb
                return 0

            lax.fori_loop(0, 8, g_body, 0)

    def chunk_step(i, b, warmup=False, do_g=True, do_l=True):
        wait_G(b)
        if do_g:
            if not warmup:
                wait_S((b + 1) % NBUF)
            wait_L(i + 1, (b + 1) % NBUF)
            issue_G((b + 1) % NBUF)
        scale(b)
        copy_dst(b)
        issue_S(b)
        if do_l:
            issue_L(i + 2, (b + 2) % NBUF)

    issue_L(0, 0)
    issue_L(1, 1)
    wait_L(0, 0)
    issue_G(0)
    chunk_step(0, 0, warmup=True)
    chunk_step(1, 1, warmup=True)

    def k_body(k, _):
        chunk_step(3 * k + 2, 2)
        chunk_step(3 * k + 3, 0)
        chunk_step(3 * k + 4, 1)
        return 0

    lax.fori_loop(0, (N_CHUNKS - 4) // 3, k_body, 0)
    chunk_step(N_CHUNKS - 2, (N_CHUNKS - 2) % NBUF, do_l=False)
    chunk_step(N_CHUNKS - 1, (N_CHUNKS - 1) % NBUF, do_g=False, do_l=False)
    wait_S((N_CHUNKS - 3) % NBUF)
    wait_S((N_CHUNKS - 2) % NBUF)
    wait_S((N_CHUNKS - 1) % NBUF)

    plsc.subcore_barrier()

    @pl.when(sid < 15)
    def _():
        pltpu.sync_copy(acc.at[pl.ds(sid * ROWS_MAIN, ROWS_MAIN)],
                        out_hbm.at[cid].at[pl.ds(sid * ROWS_MAIN, ROWS_MAIN)])

    @pl.when(sid == 15)
    def _():
        pltpu.sync_copy(acc.at[pl.ds(15 * ROWS_MAIN, ROWS_LAST)],
                        out_hbm.at[cid].at[pl.ds(15 * ROWS_MAIN, ROWS_LAST)])


def _sc_conv(y2, edata, e_flat, zeros):
    mesh = plsc.VectorSubcoreMesh(core_axis_name="c", subcore_axis_name="s")
    f = pl.kernel(
        _sc_conv_body,
        out_type=jax.ShapeDtypeStruct((2, N_NODES, 32), jnp.float32),
        mesh=mesh,
        scratch_types=(
            [pltpu.VMEM((SUB, 2, 128), jnp.int32)] * 3
            + [pltpu.VMEM((SUB, 128), jnp.int32)] * 3
            + [pltpu.VMEM((CHUNK,), jnp.float32)] * 3
            + [pltpu.VMEM((SUB, 128, 32), jnp.float32)] * 3
            + [pltpu.VMEM_SHARED((N_NODES, 32), jnp.float32),
               pltpu.SemaphoreType.DMA,
               pltpu.SemaphoreType.DMA,
               pltpu.SemaphoreType.DMA]
        ),
        compiler_params=pltpu.CompilerParams(use_tc_tiling_on_sc=False),
    )
    return f(y2, edata, e_flat, zeros)


def _mm(a, w):
    return jax.lax.dot_general(a, w, (((1,), (0,)), ((), ())),
                               preferred_element_type=jnp.float32)


def _dense1_body(x_ref, W11_ref, b11_ref, W12_ref, b12_ref, W13_ref, b13_ref,
                 Wc1_ref, a_ref, c_ref, y_ref):
    xb = x_ref[...]
    a_ref[...] = jnp.maximum(_mm(xb, W11_ref[...]) + b11_ref[...], 0.0)
    c_ref[...] = (jnp.maximum(_mm(xb, W12_ref[...]) + b12_ref[...], 0.0)
                  * jnp.maximum(_mm(xb, W13_ref[...]) + b13_ref[...], 0.0))
    y = _mm(xb, Wc1_ref[...])
    y_ref[0] = y[:, :32]
    y_ref[1] = y[:, 32:]


def _dense2_body(a_ref, c_ref, h_ref, bc1_ref,
                 W21a_ref, W21b_ref, W21c_ref, b21_ref,
                 W22a_ref, W22b_ref, W22c_ref, b22_ref,
                 W23a_ref, W23b_ref, W23c_ref, b23_ref,
                 Wc2a_ref, Wc2b_ref, Wc2c_ref,
                 p_ref, q_ref, y_ref):
    ab = a_ref[...]
    cb = c_ref[...]
    conv = jnp.maximum(
        jnp.concatenate([h_ref[0], h_ref[1]], axis=1) + bc1_ref[...], 0.0)
    p_ref[...] = jnp.maximum(
        _mm(ab, W21a_ref[...]) + _mm(conv, W21b_ref[...])
        + _mm(cb, W21c_ref[...]) + b21_ref[...], 0.0)
    q_ref[...] = (
        jnp.maximum(_mm(ab, W22a_ref[...]) + _mm(conv, W22b_ref[...])
                    + _mm(cb, W22c_ref[...]) + b22_ref[...], 0.0)
        * jnp.maximum(_mm(ab, W23a_ref[...]) + _mm(conv, W23b_ref[...])
                      + _mm(cb, W23c_ref[...]) + b23_ref[...], 0.0))
    y = (_mm(ab, Wc2a_ref[...]) + _mm(conv, Wc2b_ref[...])
         + _mm(cb, Wc2c_ref[...]))
    y_ref[0] = y[:, :32]
    y_ref[1] = y[:, 32:]


def _dense3_body(p_ref, q_ref, h_ref, bc2_ref,
                 W2a_ref, W2b_ref, W2c_ref, b2_ref, out_ref):
    conv = jnp.maximum(
        jnp.concatenate([h_ref[0], h_ref[1]], axis=1) + bc2_ref[...], 0.0)
    z = (_mm(p_ref[...], W2a_ref[...]) + _mm(conv, W2b_ref[...])
         + _mm(q_ref[...], W2c_ref[...]) + b2_ref[...])
    m = jnp.max(z, axis=1, keepdims=True)
    zs = z - m
    out_ref[...] = zs - jnp.log(jnp.sum(jnp.exp(zs), axis=1, keepdims=True))


def _row_spec(w):
    return pl.BlockSpec((BN, w), lambda i: (i, 0))


def _half_spec():
    return pl.BlockSpec((2, BN, 32), lambda i: (0, i, 0))


def _w_spec(shape):
    return pl.BlockSpec(shape, lambda i: tuple(0 for _ in shape))


def kernel(x, edge_index, edge_feats, W11, b11, Wc1, bc1, W12, b12, W13, b13,
           W21, b21, Wc2, bc2, W22, b22, W23, b23, W2, b2):
    n = x.shape[0]
    e_cnt = edge_index.shape[1]
    assert n == N_NODES and e_cnt == N_EDGES

    pad = E_ALLOC - e_cnt
    src = jnp.concatenate([edge_index[0], jnp.zeros((pad,), jnp.int32)])
    dst = jnp.concatenate(
        [edge_index[1], jnp.arange(pad, dtype=jnp.int32) % n])
    ew = jnp.concatenate([edge_feats[:, 0], jnp.zeros((pad,), jnp.float32)])
    edata = jnp.stack([src.reshape(-1, 128), dst.reshape(-1, 128)], axis=1)
    zeros = jnp.zeros((n, 32), jnp.float32)

    W21a, W21b, W21c = W21[:64], W21[64:128], W21[128:]
    W22a, W22b, W22c = W22[:64], W22[64:128], W22[128:]
    W23a, W23b, W23c = W23[:64], W23[64:128], W23[128:]
    Wc2a, Wc2b, Wc2c = Wc2[0][:64], Wc2[0][64:128], Wc2[0][128:]
    W2a, W2b, W2c = W2[:64], W2[64:128], W2[128:]
    b11r, b12r, b13r = b11[None], b12[None], b13[None]
    b21r, b22r, b23r = b21[None], b22[None], b23[None]
    bc1r, bc2r, b2r = bc1[None], bc2[None], b2[None]

    a, c, y0 = pl.pallas_call(
        _dense1_body,
        grid=(GRID,),
        in_specs=[_row_spec(64), _w_spec((64, 64)), _w_spec((1, 64)),
                  _w_spec((64, 16)), _w_spec((1, 16)),
                  _w_spec((64, 16)), _w_spec((1, 16)),
                  _w_spec((64, 64))],
        out_specs=[_row_spec(64), _row_spec(16), _half_spec()],
        out_shape=[jax.ShapeDtypeStruct((n, 64), jnp.float32),
                   jax.ShapeDtypeStruct((n, 16), jnp.float32),
                   jax.ShapeDtypeStruct((2, n, 32), jnp.float32)],
    )(x, W11, b11r, W12, b12r, W13, b13r, Wc1[0])

    h0 = _sc_conv(y0, edata, ew, zeros)

    p, q, y1 = pl.pallas_call(
        _dense2_body,
        grid=(GRID,),
        in_specs=[_row_spec(64), _row_spec(16), _half_spec(), _w_spec((1, 64)),
                  _w_spec((64, 64)), _w_spec((64, 64)), _w_spec((16, 64)),
                  _w_spec((1, 64)),
                  _w_spec((64, 16)), _w_spec((64, 16)), _w_spec((16, 16)),
                  _w_spec((1, 16)),
                  _w_spec((64, 16)), _w_spec((64, 16)), _w_spec((16, 16)),
                  _w_spec((1, 16)),
                  _w_spec((64, 64)), _w_spec((64, 64)), _w_spec((16, 64))],
        out_specs=[_row_spec(64), _row_spec(16), _half_spec()],
        out_shape=[jax.ShapeDtypeStruct((n, 64), jnp.float32),
                   jax.ShapeDtypeStruct((n, 16), jnp.float32),
                   jax.ShapeDtypeStruct((2, n, 32), jnp.float32)],
    )(a, c, h0, bc1r,
      W21a, W21b, W21c, b21r,
      W22a, W22b, W22c, b22r,
      W23a, W23b, W23c, b23r,
      Wc2a, Wc2b, Wc2c)

    h1 = _sc_conv(y1, edata, ew, zeros)

    out = pl.pallas_call(
        _dense3_body,
        grid=(GRID,),
        in_specs=[_row_spec(64), _row_spec(16), _half_spec(), _w_spec((1, 64)),
                  _w_spec((64, 128)), _w_spec((64, 128)), _w_spec((16, 128)),
                  _w_spec((1, 128))],
        out_specs=_row_spec(128),
        out_shape=jax.ShapeDtypeStruct((n, 128), jnp.float32),
    )(p, q, h1, bc2r, W2a, W2b, W2c, b2r)

    return out


# speedup vs baseline: 10.6483x; 1.0042x over previous
import functools

import jax
import jax.numpy as jnp
from jax import lax
from jax.experimental import pallas as pl
from jax.experimental.pallas import tpu as pltpu
from jax.experimental.pallas import tpu_sc as plsc

N_NODES = 50000
N_EDGES = 800000
N_SUBCORES = 16
CHUNK = 256
SUB = CHUNK // 128
N_CHUNKS = 196
NBUF = 3
EDGES_PER_TILE = N_CHUNKS * CHUNK
E_PAD = EDGES_PER_TILE * N_SUBCORES
E_ALLOC = E_PAD
ROWS_MAIN = 3128
ROWS_LAST = N_NODES - 15 * ROWS_MAIN
BN = 2000
GRID = N_NODES // BN


def _lane_bcast(v, i):
    return lax.gather(
        v, jnp.full((16, 1), i, jnp.int32),
        lax.GatherDimensionNumbers(offset_dims=(), collapsed_slice_dims=(0,),
                                   start_index_map=(0,)),
        (1,), mode=lax.GatherScatterMode.PROMISE_IN_BOUNDS)


def _sc_conv_body(y_hbm, src_hbm, dst_hbm, e_hbm, zero_hbm, out_hbm,
                  srcw0, srcw1, srcw2, dstw0, dstw1, dstw2,
                  sdst0, sdst1, sdst2, ew0, ew1, ew2,
                  rows0, rows1, rows2, acc, gsem, ssem, lsem):
    cid = lax.axis_index("c")
    sid = lax.axis_index("s")
    ytab = y_hbm.at[cid]
    srcw = (srcw0, srcw1, srcw2)
    dstw = (dstw0, dstw1, dstw2)
    ew = (ew0, ew1, ew2)
    sdst = (sdst0, sdst1, sdst2)
    rows = (rows0, rows1, rows2)

    @pl.when(sid < 15)
    def _():
        pltpu.sync_copy(zero_hbm.at[pl.ds(sid * ROWS_MAIN, ROWS_MAIN)],
                        acc.at[pl.ds(sid * ROWS_MAIN, ROWS_MAIN)])

    @pl.when(sid == 15)
    def _():
        pltpu.sync_copy(zero_hbm.at[pl.ds(15 * ROWS_MAIN, ROWS_LAST)],
                        acc.at[pl.ds(15 * ROWS_MAIN, ROWS_LAST)])

    plsc.subcore_barrier()


    def _lin(i, b):
        base = sid * EDGES_PER_TILE + i * CHUNK
        return ((src_hbm.at[pl.ds(base, CHUNK)], srcw[b]),
                (dst_hbm.at[pl.ds(base, CHUNK)], dstw[b]),
                (e_hbm.at[pl.ds(base, CHUNK)], ew[b]))

    def issue_L(i, b):
        for s, d in _lin(i, b):
            pltpu.async_copy(s, d, lsem)

    def wait_L(i, b):
        for s, d in _lin(i, b):
            pltpu.make_async_copy(s, d, lsem).wait()

    def issue_G(b):
        pltpu.async_copy(ytab.at[srcw[b]], rows[b], gsem)

    def wait_G(b):
        pltpu.make_async_copy(ytab.at[srcw[b]], rows[b], gsem).wait()

    def issue_S(b):
        pltpu.async_copy(rows[b], acc.at[sdst[b]], ssem, add=True)

    def wait_S(b):
        pltpu.make_async_copy(rows[b], acc.at[sdst[b]], ssem).wait()

    def copy_dst(b):
        for t in range(CHUNK // 16):
            sdst[b][pl.ds(t * 16, 16)] = dstw[b][pl.ds(t * 16, 16)]

    def scale(b):
        view = rows[b]

        def g_body(g, _):
            ev = ew[b][pl.ds(g * 16, 16)]
            for t in range(16):
                eb = _lane_bcast(ev, t)
                r = g * 16 + t
                view[r, pl.ds(0, 16)] = view[r, pl.ds(0, 16)] * eb
                view[r, pl.ds(16, 16)] = view[r, pl.ds(16, 16)] * eb
            return 0

        lax.fori_loop(0, CHUNK // 16, g_body, 0)

    def chunk_step(i, b, warmup=False, do_g=True, do_l=True):
        wait_G(b)
        if do_g:
            if not warmup:
                wait_S((b + 1) % NBUF)
            wait_L(i + 1, (b + 1) % NBUF)
            issue_G((b + 1) % NBUF)
        scale(b)
        copy_dst(b)
        issue_S(b)
        if do_l:
            issue_L(i + 2, (b + 2) % NBUF)

    issue_L(0, 0)
    issue_L(1, 1)
    wait_L(0, 0)
    issue_G(0)
    chunk_step(0, 0, warmup=True)
    chunk_step(1, 1, warmup=True)

    def k_body(k, _):
        chunk_step(3 * k + 2, 2)
        chunk_step(3 * k + 3, 0)
        chunk_step(3 * k + 4, 1)
        return 0

    lax.fori_loop(0, (N_CHUNKS - 4) // 3, k_body, 0)
    chunk_step(N_CHUNKS - 2, (N_CHUNKS - 2) % NBUF, do_l=False)
    chunk_step(N_CHUNKS - 1, (N_CHUNKS - 1) % NBUF, do_g=False, do_l=False)
    wait_S((N_CHUNKS - 3) % NBUF)
    wait_S((N_CHUNKS - 2) % NBUF)
    wait_S((N_CHUNKS - 1) % NBUF)

    plsc.subcore_barrier()

    @pl.when(sid < 15)
    def _():
        pltpu.sync_copy(acc.at[pl.ds(sid * ROWS_MAIN, ROWS_MAIN)],
                        out_hbm.at[cid].at[pl.ds(sid * ROWS_MAIN, ROWS_MAIN)])

    @pl.when(sid == 15)
    def _():
        pltpu.sync_copy(acc.at[pl.ds(15 * ROWS_MAIN, ROWS_LAST)],
                        out_hbm.at[cid].at[pl.ds(15 * ROWS_MAIN, ROWS_LAST)])


def _sc_conv(y2, src_flat, dst_flat, e_flat, zeros):
    mesh = plsc.VectorSubcoreMesh(core_axis_name="c", subcore_axis_name="s")
    f = pl.kernel(
        _sc_conv_body,
        out_type=jax.ShapeDtypeStruct((2, N_NODES, 32), jnp.float32),
        mesh=mesh,
        scratch_types=(
            [pltpu.VMEM((CHUNK,), jnp.int32)] * 9
            + [pltpu.VMEM((CHUNK,), jnp.float32)] * 3
            + [pltpu.VMEM((CHUNK, 32), jnp.float32)] * 3
            + [pltpu.VMEM_SHARED((N_NODES, 32), jnp.float32),
               pltpu.SemaphoreType.DMA,
               pltpu.SemaphoreType.DMA,
               pltpu.SemaphoreType.DMA]
        ),
        compiler_params=pltpu.CompilerParams(use_tc_tiling_on_sc=False),
    )
    return f(y2, src_flat, dst_flat, e_flat, zeros)


def _mm(a, w):
    return jax.lax.dot_general(a, w, (((1,), (0,)), ((), ())),
                               preferred_element_type=jnp.float32)


def _dense1_body(x_ref, W11_ref, b11_ref, W12_ref, b12_ref, W13_ref, b13_ref,
                 Wc1_ref, a_ref, c_ref, y_ref):
    xb = x_ref[...]
    a_ref[...] = jnp.maximum(_mm(xb, W11_ref[...]) + b11_ref[...], 0.0)
    c_ref[...] = (jnp.maximum(_mm(xb, W12_ref[...]) + b12_ref[...], 0.0)
                  * jnp.maximum(_mm(xb, W13_ref[...]) + b13_ref[...], 0.0))
    y = _mm(xb, Wc1_ref[...])
    y_ref[0] = y[:, :32]
    y_ref[1] = y[:, 32:]


def _dense2_body(a_ref, c_ref, h_ref, bc1_ref,
                 W21a_ref, W21b_ref, W21c_ref, b21_ref,
                 W22a_ref, W22b_ref, W22c_ref, b22_ref,
                 W23a_ref, W23b_ref, W23c_ref, b23_ref,
                 Wc2a_ref, Wc2b_ref, Wc2c_ref,
                 p_ref, q_ref, y_ref):
    ab = a_ref[...]
    cb = c_ref[...]
    conv = jnp.maximum(
        jnp.concatenate([h_ref[0], h_ref[1]], axis=1) + bc1_ref[...], 0.0)
    p_ref[...] = jnp.maximum(
        _mm(ab, W21a_ref[...]) + _mm(conv, W21b_ref[...])
        + _mm(cb, W21c_ref[...]) + b21_ref[...], 0.0)
    q_ref[...] = (
        jnp.maximum(_mm(ab, W22a_ref[...]) + _mm(conv, W22b_ref[...])
                    + _mm(cb, W22c_ref[...]) + b22_ref[...], 0.0)
        * jnp.maximum(_mm(ab, W23a_ref[...]) + _mm(conv, W23b_ref[...])
                      + _mm(cb, W23c_ref[...]) + b23_ref[...], 0.0))
    y = (_mm(ab, Wc2a_ref[...]) + _mm(conv, Wc2b_ref[...])
         + _mm(cb, Wc2c_ref[...]))
    y_ref[0] = y[:, :32]
    y_ref[1] = y[:, 32:]


def _dense3_body(p_ref, q_ref, h_ref, bc2_ref,
                 W2a_ref, W2b_ref, W2c_ref, b2_ref, out_ref):
    conv = jnp.maximum(
        jnp.concatenate([h_ref[0], h_ref[1]], axis=1) + bc2_ref[...], 0.0)
    z = (_mm(p_ref[...], W2a_ref[...]) + _mm(conv, W2b_ref[...])
         + _mm(q_ref[...], W2c_ref[...]) + b2_ref[...])
    m = jnp.max(z, axis=1, keepdims=True)
    zs = z - m
    out_ref[...] = zs - jnp.log(jnp.sum(jnp.exp(zs), axis=1, keepdims=True))


def _row_spec(w):
    return pl.BlockSpec((BN, w), lambda i: (i, 0))


def _half_spec():
    return pl.BlockSpec((2, BN, 32), lambda i: (0, i, 0))


def _w_spec(shape):
    return pl.BlockSpec(shape, lambda i: tuple(0 for _ in shape))


def kernel(x, edge_index, edge_feats, W11, b11, Wc1, bc1, W12, b12, W13, b13,
           W21, b21, Wc2, bc2, W22, b22, W23, b23, W2, b2):
    n = x.shape[0]
    e_cnt = edge_index.shape[1]
    assert n == N_NODES and e_cnt == N_EDGES

    pad = E_ALLOC - e_cnt
    src = jnp.concatenate([edge_index[0], jnp.zeros((pad,), jnp.int32)])
    dst = jnp.concatenate(
        [edge_index[1], jnp.arange(pad, dtype=jnp.int32) % n])
    ew = jnp.concatenate([edge_feats[:, 0], jnp.zeros((pad,), jnp.float32)])
    zeros = jnp.zeros((n, 32), jnp.float32)

    W21a, W21b, W21c = W21[:64], W21[64:128], W21[128:]
    W22a, W22b, W22c = W22[:64], W22[64:128], W22[128:]
    W23a, W23b, W23c = W23[:64], W23[64:128], W23[128:]
    Wc2a, Wc2b, Wc2c = Wc2[0][:64], Wc2[0][64:128], Wc2[0][128:]
    W2a, W2b, W2c = W2[:64], W2[64:128], W2[128:]
    b11r, b12r, b13r = b11[None], b12[None], b13[None]
    b21r, b22r, b23r = b21[None], b22[None], b23[None]
    bc1r, bc2r, b2r = bc1[None], bc2[None], b2[None]

    a, c, y0 = pl.pallas_call(
        _dense1_body,
        grid=(GRID,),
        in_specs=[_row_spec(64), _w_spec((64, 64)), _w_spec((1, 64)),
                  _w_spec((64, 16)), _w_spec((1, 16)),
                  _w_spec((64, 16)), _w_spec((1, 16)),
                  _w_spec((64, 64))],
        out_specs=[_row_spec(64), _row_spec(16), _half_spec()],
        out_shape=[jax.ShapeDtypeStruct((n, 64), jnp.float32),
                   jax.ShapeDtypeStruct((n, 16), jnp.float32),
                   jax.ShapeDtypeStruct((2, n, 32), jnp.float32)],
    )(x, W11, b11r, W12, b12r, W13, b13r, Wc1[0])

    h0 = _sc_conv(y0, src, dst, ew, zeros)

    p, q, y1 = pl.pallas_call(
        _dense2_body,
        grid=(GRID,),
        in_specs=[_row_spec(64), _row_spec(16), _half_spec(), _w_spec((1, 64)),
                  _w_spec((64, 64)), _w_spec((64, 64)), _w_spec((16, 64)),
                  _w_spec((1, 64)),
                  _w_spec((64, 16)), _w_spec((64, 16)), _w_spec((16, 16)),
                  _w_spec((1, 16)),
                  _w_spec((64, 16)), _w_spec((64, 16)), _w_spec((16, 16)),
                  _w_spec((1, 16)),
                  _w_spec((64, 64)), _w_spec((64, 64)), _w_spec((16, 64))],
        out_specs=[_row_spec(64), _row_spec(16), _half_spec()],
        out_shape=[jax.ShapeDtypeStruct((n, 64), jnp.float32),
                   jax.ShapeDtypeStruct((n, 16), jnp.float32),
                   jax.ShapeDtypeStruct((2, n, 32), jnp.float32)],
    )(a, c, h0, bc1r,
      W21a, W21b, W21c, b21r,
      W22a, W22b, W22c, b22r,
      W23a, W23b, W23c, b23r,
      Wc2a, Wc2b, Wc2c)

    h1 = _sc_conv(y1, src, dst, ew, zeros)

    out = pl.pallas_call(
        _dense3_body,
        grid=(GRID,),
        in_specs=[_row_spec(64), _row_spec(16), _half_spec(), _w_spec((1, 64)),
                  _w_spec((64, 128)), _w_spec((64, 128)), _w_spec((16, 128)),
                  _w_spec((1, 128))],
        out_specs=_row_spec(128),
        out_shape=jax.ShapeDtypeStruct((n, 128), jnp.float32),
    )(p, q, h1, bc2r, W2a, W2b, W2c, b2r)

    return out


# speedup vs baseline: 10.6661x; 1.0017x over previous
import functools

import jax
import jax.numpy as jnp
from jax import lax
from jax.experimental import pallas as pl
from jax.experimental.pallas import tpu as pltpu
from jax.experimental.pallas import tpu_sc as plsc

N_NODES = 50000
N_EDGES = 800000
N_SUBCORES = 16
CHUNK = 256
SUB = CHUNK // 128
N_CHUNKS = 196
NBUF = 3
EDGES_PER_TILE = N_CHUNKS * CHUNK
E_PAD = EDGES_PER_TILE * N_SUBCORES
E_ALLOC = E_PAD
ROWS_MAIN = 3128
ROWS_LAST = N_NODES - 15 * ROWS_MAIN
BN = 2000
GRID = N_NODES // BN


def _lane_bcast(v, i):
    return lax.gather(
        v, jnp.full((16, 1), i, jnp.int32),
        lax.GatherDimensionNumbers(offset_dims=(), collapsed_slice_dims=(0,),
                                   start_index_map=(0,)),
        (1,), mode=lax.GatherScatterMode.PROMISE_IN_BOUNDS)


def _sc_conv_body(y_hbm, src_hbm, dst_hbm, e_hbm, zero_hbm, out_hbm,
                  srcw0, srcw1, srcw2, dstw0, dstw1, dstw2,
                  sdst0, sdst1, sdst2, ew0, ew1, ew2,
                  rows0, rows1, rows2, acc, gsem, ssem, lsem):
    cid = lax.axis_index("c")
    sid = lax.axis_index("s")
    ytab = y_hbm.at[cid]
    srcw = (srcw0, srcw1, srcw2)
    dstw = (dstw0, dstw1, dstw2)
    ew = (ew0, ew1, ew2)
    sdst = (sdst0, sdst1, sdst2)
    rows = (rows0, rows1, rows2)

    @pl.when(sid < 15)
    def _():
        pltpu.sync_copy(zero_hbm.at[pl.ds(sid * ROWS_MAIN, ROWS_MAIN)],
                        acc.at[pl.ds(sid * ROWS_MAIN, ROWS_MAIN)])

    @pl.when(sid == 15)
    def _():
        pltpu.sync_copy(zero_hbm.at[pl.ds(15 * ROWS_MAIN, ROWS_LAST)],
                        acc.at[pl.ds(15 * ROWS_MAIN, ROWS_LAST)])

    plsc.subcore_barrier()


    def _lin(i, b):
        base = sid * EDGES_PER_TILE + i * CHUNK
        return ((src_hbm.at[pl.ds(base, CHUNK)], srcw[b]),
                (dst_hbm.at[pl.ds(base, CHUNK)], dstw[b]),
                (e_hbm.at[pl.ds(base, CHUNK)], ew[b]))

    def issue_L(i, b):
        for s, d in _lin(i, b):
            pltpu.async_copy(s, d, lsem)

    def wait_L(i, b):
        for s, d in _lin(i, b):
            pltpu.make_async_copy(s, d, lsem).wait()

    def issue_G(b):
        pltpu.async_copy(ytab.at[srcw[b]], rows[b], gsem)

    def wait_G(b):
        pltpu.make_async_copy(ytab.at[srcw[b]], rows[b], gsem).wait()

    def issue_S(b):
        pltpu.async_copy(rows[b], acc.at[sdst[b]], ssem, add=True)

    def wait_S(b):
        pltpu.make_async_copy(rows[b], acc.at[sdst[b]], ssem).wait()

    def copy_dst(b):
        for t in range(CHUNK // 16):
            sdst[b][pl.ds(t * 16, 16)] = dstw[b][pl.ds(t * 16, 16)]

    def scale(b):
        view = rows[b]

        def g_body(g, _):
            ev = ew[b][pl.ds(g * 16, 16)]
            for t in range(16):
                eb = _lane_bcast(ev, t)
                r = g * 16 + t
                view[r, pl.ds(0, 16)] = view[r, pl.ds(0, 16)] * eb
                view[r, pl.ds(16, 16)] = view[r, pl.ds(16, 16)] * eb
            return 0

        lax.fori_loop(0, CHUNK // 16, g_body, 0)

    def chunk_step(i, b, warmup=False, do_g=True, do_l=True):
        wait_G(b)
        if do_g:
            if not warmup:
                wait_S((b + 2) % NBUF)
            wait_L(i + 1, (b + 1) % NBUF)
            issue_G((b + 1) % NBUF)
        scale(b)
        copy_dst(b)
        issue_S(b)
        if do_l:
            issue_L(i + 2, (b + 2) % NBUF)

    issue_L(0, 0)
    issue_L(1, 1)
    wait_L(0, 0)
    issue_G(0)
    chunk_step(0, 0, warmup=True)
    chunk_step(1, 1)

    def k_body(k, _):
        chunk_step(3 * k + 2, 2)
        chunk_step(3 * k + 3, 0)
        chunk_step(3 * k + 4, 1)
        return 0

    lax.fori_loop(0, (N_CHUNKS - 4) // 3, k_body, 0)
    chunk_step(N_CHUNKS - 2, (N_CHUNKS - 2) % NBUF, do_l=False)
    chunk_step(N_CHUNKS - 1, (N_CHUNKS - 1) % NBUF, do_g=False, do_l=False)
    wait_S((N_CHUNKS - 2) % NBUF)
    wait_S((N_CHUNKS - 1) % NBUF)

    plsc.subcore_barrier()

    @pl.when(sid < 15)
    def _():
        pltpu.sync_copy(acc.at[pl.ds(sid * ROWS_MAIN, ROWS_MAIN)],
                        out_hbm.at[cid].at[pl.ds(sid * ROWS_MAIN, ROWS_MAIN)])

    @pl.when(sid == 15)
    def _():
        pltpu.sync_copy(acc.at[pl.ds(15 * ROWS_MAIN, ROWS_LAST)],
                        out_hbm.at[cid].at[pl.ds(15 * ROWS_MAIN, ROWS_LAST)])


def _sc_conv(y2, src_flat, dst_flat, e_flat, zeros):
    mesh = plsc.VectorSubcoreMesh(core_axis_name="c", subcore_axis_name="s")
    f = pl.kernel(
        _sc_conv_body,
        out_type=jax.ShapeDtypeStruct((2, N_NODES, 32), jnp.float32),
        mesh=mesh,
        scratch_types=(
            [pltpu.VMEM((CHUNK,), jnp.int32)] * 9
            + [pltpu.VMEM((CHUNK,), jnp.float32)] * 3
            + [pltpu.VMEM((CHUNK, 32), jnp.float32)] * 3
            + [pltpu.VMEM_SHARED((N_NODES, 32), jnp.float32),
               pltpu.SemaphoreType.DMA,
               pltpu.SemaphoreType.DMA,
               pltpu.SemaphoreType.DMA]
        ),
        compiler_params=pltpu.CompilerParams(use_tc_tiling_on_sc=False),
    )
    return f(y2, src_flat, dst_flat, e_flat, zeros)


def _mm(a, w):
    return jax.lax.dot_general(a, w, (((1,), (0,)), ((), ())),
                               preferred_element_type=jnp.float32)


def _dense1_body(x_ref, W11_ref, b11_ref, W12_ref, b12_ref, W13_ref, b13_ref,
                 Wc1_ref, a_ref, c_ref, y_ref):
    xb = x_ref[...]
    a_ref[...] = jnp.maximum(_mm(xb, W11_ref[...]) + b11_ref[...], 0.0)
    c_ref[...] = (jnp.maximum(_mm(xb, W12_ref[...]) + b12_ref[...], 0.0)
                  * jnp.maximum(_mm(xb, W13_ref[...]) + b13_ref[...], 0.0))
    y = _mm(xb, Wc1_ref[...])
    y_ref[0] = y[:, :32]
    y_ref[1] = y[:, 32:]


def _dense2_body(a_ref, c_ref, h_ref, bc1_ref,
                 W21a_ref, W21b_ref, W21c_ref, b21_ref,
                 W22a_ref, W22b_ref, W22c_ref, b22_ref,
                 W23a_ref, W23b_ref, W23c_ref, b23_ref,
                 Wc2a_ref, Wc2b_ref, Wc2c_ref,
                 p_ref, q_ref, y_ref):
    ab = a_ref[...]
    cb = c_ref[...]
    conv = jnp.maximum(
        jnp.concatenate([h_ref[0], h_ref[1]], axis=1) + bc1_ref[...], 0.0)
    p_ref[...] = jnp.maximum(
        _mm(ab, W21a_ref[...]) + _mm(conv, W21b_ref[...])
        + _mm(cb, W21c_ref[...]) + b21_ref[...], 0.0)
    q_ref[...] = (
        jnp.maximum(_mm(ab, W22a_ref[...]) + _mm(conv, W22b_ref[...])
                    + _mm(cb, W22c_ref[...]) + b22_ref[...], 0.0)
        * jnp.maximum(_mm(ab, W23a_ref[...]) + _mm(conv, W23b_ref[...])
                      + _mm(cb, W23c_ref[...]) + b23_ref[...], 0.0))
    y = (_mm(ab, Wc2a_ref[...]) + _mm(conv, Wc2b_ref[...])
         + _mm(cb, Wc2c_ref[...]))
    y_ref[0] = y[:, :32]
    y_ref[1] = y[:, 32:]


def _dense3_body(p_ref, q_ref, h_ref, bc2_ref,
                 W2a_ref, W2b_ref, W2c_ref, b2_ref, out_ref):
    conv = jnp.maximum(
        jnp.concatenate([h_ref[0], h_ref[1]], axis=1) + bc2_ref[...], 0.0)
    z = (_mm(p_ref[...], W2a_ref[...]) + _mm(conv, W2b_ref[...])
         + _mm(q_ref[...], W2c_ref[...]) + b2_ref[...])
    m = jnp.max(z, axis=1, keepdims=True)
    zs = z - m
    out_ref[...] = zs - jnp.log(jnp.sum(jnp.exp(zs), axis=1, keepdims=True))


def _row_spec(w):
    return pl.BlockSpec((BN, w), lambda i: (i, 0))


def _half_spec():
    return pl.BlockSpec((2, BN, 32), lambda i: (0, i, 0))


def _w_spec(shape):
    return pl.BlockSpec(shape, lambda i: tuple(0 for _ in shape))


def kernel(x, edge_index, edge_feats, W11, b11, Wc1, bc1, W12, b12, W13, b13,
           W21, b21, Wc2, bc2, W22, b22, W23, b23, W2, b2):
    n = x.shape[0]
    e_cnt = edge_index.shape[1]
    assert n == N_NODES and e_cnt == N_EDGES

    pad = E_ALLOC - e_cnt
    src = jnp.concatenate([edge_index[0], jnp.zeros((pad,), jnp.int32)])
    dst = jnp.concatenate(
        [edge_index[1], jnp.arange(pad, dtype=jnp.int32) % n])
    ew = jnp.concatenate([edge_feats[:, 0], jnp.zeros((pad,), jnp.float32)])
    zeros = jnp.zeros((n, 32), jnp.float32)

    W21a, W21b, W21c = W21[:64], W21[64:128], W21[128:]
    W22a, W22b, W22c = W22[:64], W22[64:128], W22[128:]
    W23a, W23b, W23c = W23[:64], W23[64:128], W23[128:]
    Wc2a, Wc2b, Wc2c = Wc2[0][:64], Wc2[0][64:128], Wc2[0][128:]
    W2a, W2b, W2c = W2[:64], W2[64:128], W2[128:]
    b11r, b12r, b13r = b11[None], b12[None], b13[None]
    b21r, b22r, b23r = b21[None], b22[None], b23[None]
    bc1r, bc2r, b2r = bc1[None], bc2[None], b2[None]

    a, c, y0 = pl.pallas_call(
        _dense1_body,
        grid=(GRID,),
        in_specs=[_row_spec(64), _w_spec((64, 64)), _w_spec((1, 64)),
                  _w_spec((64, 16)), _w_spec((1, 16)),
                  _w_spec((64, 16)), _w_spec((1, 16)),
                  _w_spec((64, 64))],
        out_specs=[_row_spec(64), _row_spec(16), _half_spec()],
        out_shape=[jax.ShapeDtypeStruct((n, 64), jnp.float32),
                   jax.ShapeDtypeStruct((n, 16), jnp.float32),
                   jax.ShapeDtypeStruct((2, n, 32), jnp.float32)],
    )(x, W11, b11r, W12, b12r, W13, b13r, Wc1[0])

    h0 = _sc_conv(y0, src, dst, ew, zeros)

    p, q, y1 = pl.pallas_call(
        _dense2_body,
        grid=(GRID,),
        in_specs=[_row_spec(64), _row_spec(16), _half_spec(), _w_spec((1, 64)),
                  _w_spec((64, 64)), _w_spec((64, 64)), _w_spec((16, 64)),
                  _w_spec((1, 64)),
                  _w_spec((64, 16)), _w_spec((64, 16)), _w_spec((16, 16)),
                  _w_spec((1, 16)),
                  _w_spec((64, 16)), _w_spec((64, 16)), _w_spec((16, 16)),
                  _w_spec((1, 16)),
                  _w_spec((64, 64)), _w_spec((64, 64)), _w_spec((16, 64))],
        out_specs=[_row_spec(64), _row_spec(16), _half_spec()],
        out_shape=[jax.ShapeDtypeStruct((n, 64), jnp.float32),
                   jax.ShapeDtypeStruct((n, 16), jnp.float32),
                   jax.ShapeDtypeStruct((2, n, 32), jnp.float32)],
    )(a, c, h0, bc1r,
      W21a, W21b, W21c, b21r,
      W22a, W22b, W22c, b22r,
      W23a, W23b, W23c, b23r,
      Wc2a, Wc2b, Wc2c)

    h1 = _sc_conv(y1, src, dst, ew, zeros)

    out = pl.pallas_call(
        _dense3_body,
        grid=(GRID,),
        in_specs=[_row_spec(64), _row_spec(16), _half_spec(), _w_spec((1, 64)),
                  _w_spec((64, 128)), _w_spec((64, 128)), _w_spec((16, 128)),
                  _w_spec((1, 128))],
        out_specs=_row_spec(128),
        out_shape=jax.ShapeDtypeStruct((n, 128), jnp.float32),
    )(p, q, h1, bc2r, W2a, W2b, W2c, b2r)

    return out
